# Initial kernel scaffold; baseline (speedup 1.0000x reference)
import jax
import jax.numpy as jnp
from jax.experimental import pallas as pl


def kernel(X, Ax, W1, as1, ad1, b1, g1, be1, W2, as2, ad2, b2, g2, be2, Wih1, Whh1, bih1, bhh1, Wih2, Whh2, bih2, bhh2, L1w, L1b, L2w, L2b):
    raise NotImplementedError("write your pallas kernel here")


# speedup vs baseline: 1.6443x; 1.6443x over previous
import jax
import jax.numpy as jnp
from jax import lax
from jax.experimental import pallas as pl
from jax.experimental.pallas import tpu as pltpu

T, B, N = 12, 4, 400
H = 128
G = T * B
ROWS = G * N
BATCH = B * N


def _gat_kernel(x_ref, a_ref, w_ref, asr_ref, adc_ref, b_ref,
                y_ref, s_ref, q_ref):
    x = x_ref[0]
    a = a_ref[0]
    h = jnp.dot(x, w_ref[...], preferred_element_type=jnp.float32)
    hd = jnp.dot(h, adc_ref[...], preferred_element_type=jnp.float32)
    hs = lax.dot_general(asr_ref[...], h, (((1,), (1,)), ((), ())),
                         preferred_element_type=jnp.float32)
    e = hd + hs
    e = jnp.where(e > 0, e, 0.2 * e)
    at = a.T
    e = jnp.where(at != 0, e, -1e9)
    m = jnp.max(e, axis=1, keepdims=True)
    p = jnp.exp(e - m)
    s = jnp.sum(p, axis=1, keepdims=True)
    w = (p / s) * at
    out = jnp.dot(w, h, preferred_element_type=jnp.float32) + b_ref[...]
    y = jnp.maximum(out, 0.0)
    y_ref[0] = y
    s_ref[0] = jnp.sum(y, axis=0, keepdims=True)
    q_ref[0] = jnp.sum(y * y, axis=0, keepdims=True)


def _gru_head_kernel(y_ref, s_ref, q_ref, g_ref, be_ref,
                     wih1_ref, whh1_ref, bih1_ref, bhh1_ref,
                     wih2_ref, whh2_ref, bih2_ref, bhh2_ref,
                     l1w_ref, l1b_ref, l2w_ref, l2b_ref, o_ref):
    tot = jnp.sum(s_ref[...], axis=0)
    tot2 = jnp.sum(q_ref[...], axis=0)
    mean = tot * (1.0 / ROWS)
    var = tot2 * (1.0 / ROWS) - mean * mean
    rstd = lax.rsqrt(var + 1e-5)
    scale = g_ref[...] * rstd
    shift = be_ref[...] - mean * scale

    wih1 = wih1_ref[...]
    whh1 = whh1_ref[...]
    wih2 = wih2_ref[...]
    whh2 = whh2_ref[...]
    bih1 = bih1_ref[...]
    bhh1 = bhh1_ref[...]
    bih2 = bih2_ref[...]
    bhh2 = bhh2_ref[...]

    def gru_step(x, hprev, wih, whh, bi, bh):
        gi = jnp.dot(x, wih, preferred_element_type=jnp.float32) + bi
        gh = jnp.dot(hprev, whh, preferred_element_type=jnp.float32) + bh
        r = jax.nn.sigmoid(gi[:, :H] + gh[:, :H])
        z = jax.nn.sigmoid(gi[:, H:2 * H] + gh[:, H:2 * H])
        n = jnp.tanh(gi[:, 2 * H:] + r * gh[:, 2 * H:])
        return (1.0 - z) * n + z * hprev

    h1 = jnp.zeros((BATCH, H), jnp.float32)
    h2 = jnp.zeros((BATCH, H), jnp.float32)
    for t in range(T):
        x = y_ref[t] * scale + shift
        h1 = gru_step(x, h1, wih1, whh1, bih1, bhh1)
        h2 = gru_step(h1, h2, wih2, whh2, bih2, bhh2)

    z = jnp.concatenate([h1, h2], axis=1)
    u = jnp.dot(z, l1w_ref[...], preferred_element_type=jnp.float32)
    u = jnp.maximum(u + l1b_ref[...], 0.0)
    o = jnp.dot(u, l2w_ref[...], preferred_element_type=jnp.float32)
    o_ref[...] = jnp.maximum(o + l2b_ref[...], 0.0)


def kernel(X, Ax, W1, as1, ad1, b1, g1, be1, W2, as2, ad2, b2, g2, be2,
           Wih1, Whh1, bih1, bhh1, Wih2, Whh2, bih2, bhh2, L1w, L1b, L2w, L2b):
    f_in = X.shape[-1]
    y, s, q = pl.pallas_call(
        _gat_kernel,
        grid=(G,),
        in_specs=[
            pl.BlockSpec((1, N, f_in), lambda g: (g, 0, 0)),
            pl.BlockSpec((1, N, N), lambda g: (g, 0, 0)),
            pl.BlockSpec((f_in, H), lambda g: (0, 0)),
            pl.BlockSpec((1, H), lambda g: (0, 0)),
            pl.BlockSpec((H, 1), lambda g: (0, 0)),
            pl.BlockSpec((1, H), lambda g: (0, 0)),
        ],
        out_specs=[
            pl.BlockSpec((1, N, H), lambda g: (g, 0, 0)),
            pl.BlockSpec((1, 1, H), lambda g: (g, 0, 0)),
            pl.BlockSpec((1, 1, H), lambda g: (g, 0, 0)),
        ],
        out_shape=[
            jax.ShapeDtypeStruct((G, N, H), jnp.float32),
            jax.ShapeDtypeStruct((G, 1, H), jnp.float32),
            jax.ShapeDtypeStruct((G, 1, H), jnp.float32),
        ],
        compiler_params=pltpu.CompilerParams(
            dimension_semantics=("parallel",),
        ),
    )(X, Ax, W2, as2.reshape(1, H), ad2.reshape(H, 1), b2.reshape(1, H))

    yr = y.reshape(T, BATCH, H)
    out = pl.pallas_call(
        _gru_head_kernel,
        out_shape=jax.ShapeDtypeStruct((BATCH, 2), jnp.float32),
        compiler_params=pltpu.CompilerParams(
            vmem_limit_bytes=100 * 1024 * 1024,
        ),
    )(yr, s, q, g2.reshape(1, H), be2.reshape(1, H),
      Wih1.T, Whh1.T, bih1.reshape(1, 3 * H), bhh1.reshape(1, 3 * H),
      Wih2.T, Whh2.T, bih2.reshape(1, 3 * H), bhh2.reshape(1, 3 * H),
      L1w.T, L1b.reshape(1, H), L2w.T, L2b.reshape(1, 2))

    return out.reshape(B, N, 2)


# speedup vs baseline: 1.7536x; 1.0665x over previous
import jax
import jax.numpy as jnp
from jax import lax
from jax.experimental import pallas as pl
from jax.experimental.pallas import tpu as pltpu

T, B, N = 12, 4, 400
H = 128
G = T * B
ROWS = G * N
BATCH = B * N

_DN_T = (((1,), (1,)), ((), ()))
_DN_0 = (((0,), (0,)), ((), ()))


def _fused_kernel(x_ref, a_ref, w2_ref, as2_ref, ad2_ref, b2_ref,
                  g2_ref, be2_ref,
                  wih1_ref, whh1_ref, bih1_ref, bhh1_ref,
                  wih2_ref, whh2_ref, bih2_ref, bhh2_ref,
                  l1w_ref, l1b_ref, l2w_ref, l2b_ref,
                  o_ref, y_s, s_s, q_s):
    g = pl.program_id(0)

    @pl.when(g == 0)
    def _init():
        s_s[...] = jnp.zeros_like(s_s)
        q_s[...] = jnp.zeros_like(q_s)

    @pl.when(g < G)
    def _gat():
        x = x_ref[0]
        a = a_ref[0]
        h = jnp.dot(x, w2_ref[...], preferred_element_type=jnp.float32)
        hd = jnp.dot(h, ad2_ref[...], preferred_element_type=jnp.float32)
        hs = lax.dot_general(as2_ref[...], h, _DN_T,
                             preferred_element_type=jnp.float32)
        e = hd + hs
        e = jnp.where(e > 0, e, 0.2 * e)
        at = a.T
        e = jnp.where(at != 0, e, -1e9)
        m = jnp.max(e, axis=1, keepdims=True)
        p = jnp.exp(e - m)
        s = jnp.sum(p, axis=1, keepdims=True)
        w = (p / s) * at
        out = jnp.dot(w, h, preferred_element_type=jnp.float32)
        y = jnp.maximum(out + b2_ref[...], 0.0)
        y_s[pl.ds(g * N, N), :] = y
        s_s[...] += jnp.sum(y, axis=0, keepdims=True)
        q_s[...] += jnp.sum(y * y, axis=0, keepdims=True)

    @pl.when(g == G)
    def _gru():
        mean = s_s[...] * (1.0 / ROWS)
        var = q_s[...] * (1.0 / ROWS) - mean * mean
        rstd = lax.rsqrt(var + 1e-5)
        scale = g2_ref[...] * rstd
        shift = be2_ref[...] - mean * scale

        wih1 = wih1_ref[...]
        whh1 = whh1_ref[...]
        wih2 = wih2_ref[...]
        whh2 = whh2_ref[...]
        bih1 = bih1_ref[...]
        bhh1 = bhh1_ref[...]
        bih2 = bih2_ref[...]
        bhh2 = bhh2_ref[...]

        def gru_step(x, hprev, wih, whh, bi, bh):
            gi = jnp.dot(x, wih, preferred_element_type=jnp.float32) + bi
            gh = jnp.dot(hprev, whh, preferred_element_type=jnp.float32) + bh
            r = jax.nn.sigmoid(gi[:, :H] + gh[:, :H])
            z = jax.nn.sigmoid(gi[:, H:2 * H] + gh[:, H:2 * H])
            n = jnp.tanh(gi[:, 2 * H:] + r * gh[:, 2 * H:])
            return (1.0 - z) * n + z * hprev

        h1 = jnp.zeros((BATCH, H), jnp.float32)
        h2 = jnp.zeros((BATCH, H), jnp.float32)
        for t in range(T):
            x = y_s[t * BATCH:(t + 1) * BATCH, :] * scale + shift
            h1 = gru_step(x, h1, wih1, whh1, bih1, bhh1)
            h2 = gru_step(h1, h2, wih2, whh2, bih2, bhh2)

        z = jnp.concatenate([h1, h2], axis=1)
        u = jnp.dot(z, l1w_ref[...], preferred_element_type=jnp.float32)
        u = jnp.maximum(u + l1b_ref[...], 0.0)
        o = jnp.dot(u, l2w_ref[...], preferred_element_type=jnp.float32)
        o_ref[...] = jnp.maximum(o + l2b_ref[...], 0.0)


def kernel(X, Ax, W1, as1, ad1, b1, g1, be1, W2, as2, ad2, b2, g2, be2,
           Wih1, Whh1, bih1, bhh1, Wih2, Whh2, bih2, bhh2, L1w, L1b, L2w, L2b):
    f_in = X.shape[-1]
    graph_ix = lambda g: (jnp.minimum(g, G - 1), 0, 0)
    const_ix2 = lambda g: (0, 0)
    out = pl.pallas_call(
        _fused_kernel,
        grid=(G + 1,),
        in_specs=[
            pl.BlockSpec((1, N, f_in), graph_ix),
            pl.BlockSpec((1, N, N), graph_ix),
            pl.BlockSpec((f_in, H), const_ix2),
            pl.BlockSpec((1, H), const_ix2),
            pl.BlockSpec((H, 1), const_ix2),
            pl.BlockSpec((1, H), const_ix2),
            pl.BlockSpec((1, H), const_ix2),
            pl.BlockSpec((1, H), const_ix2),
            pl.BlockSpec((H, 3 * H), const_ix2),
            pl.BlockSpec((H, 3 * H), const_ix2),
            pl.BlockSpec((1, 3 * H), const_ix2),
            pl.BlockSpec((1, 3 * H), const_ix2),
            pl.BlockSpec((H, 3 * H), const_ix2),
            pl.BlockSpec((H, 3 * H), const_ix2),
            pl.BlockSpec((1, 3 * H), const_ix2),
            pl.BlockSpec((1, 3 * H), const_ix2),
            pl.BlockSpec((2 * H, H), const_ix2),
            pl.BlockSpec((1, H), const_ix2),
            pl.BlockSpec((H, 2), const_ix2),
            pl.BlockSpec((1, 2), const_ix2),
        ],
        out_specs=pl.BlockSpec((BATCH, 2), const_ix2),
        out_shape=jax.ShapeDtypeStruct((BATCH, 2), jnp.float32),
        scratch_shapes=[
            pltpu.VMEM((ROWS, H), jnp.float32),
            pltpu.VMEM((1, H), jnp.float32),
            pltpu.VMEM((1, H), jnp.float32),
        ],
        compiler_params=pltpu.CompilerParams(
            vmem_limit_bytes=100 * 1024 * 1024,
        ),
    )(X, Ax, W2, as2.reshape(1, H), ad2.reshape(H, 1), b2.reshape(1, H),
      g2.reshape(1, H), be2.reshape(1, H),
      Wih1.T, Whh1.T, bih1.reshape(1, 3 * H), bhh1.reshape(1, 3 * H),
      Wih2.T, Whh2.T, bih2.reshape(1, 3 * H), bhh2.reshape(1, 3 * H),
      L1w.T, L1b.reshape(1, H), L2w.T, L2b.reshape(1, 2))

    return out.reshape(B, N, 2)


# speedup vs baseline: 2.3394x; 1.3341x over previous
import jax
import jax.numpy as jnp
from jax import lax
from jax.experimental import pallas as pl
from jax.experimental.pallas import tpu as pltpu

T, B, N = 12, 4, 400
H = 128
G = T * B
ROWS = G * N
BATCH = B * N

_DN_RT = (((1,), (1,)), ((), ()))


def _fused_kernel(x_ref, a_ref, w2_ref, as2_ref, ad2_ref, b2_ref,
                  g2_ref, be2_ref,
                  wih1_ref, whh1_ref, bih1_ref, bhh1_ref,
                  wih2_ref, whh2_ref, bih2_ref, bhh2_ref,
                  l1w_ref, l1b_ref, l2w_ref, l2b_ref,
                  o_ref, y_s, s_s, q_s):
    g = pl.program_id(0)

    @pl.when(g == 0)
    def _init():
        s_s[...] = jnp.zeros_like(s_s)
        q_s[...] = jnp.zeros_like(q_s)

    @pl.when(g < T)
    def _gat():
        ssum = None
        qsum = None
        for k in range(B):
            x = x_ref[0, k]
            a = a_ref[0, k]
            h = jnp.dot(x, w2_ref[...], preferred_element_type=jnp.float32)
            hd = jnp.dot(h, ad2_ref[...],
                         preferred_element_type=jnp.float32)
            hs = lax.dot_general(as2_ref[...], h, _DN_RT,
                                 preferred_element_type=jnp.float32)
            e = hd + hs
            e = jnp.maximum(e, 0.2 * e)
            at = a.T
            m = jnp.max(e, axis=1, keepdims=True)
            p = jnp.exp(e - m)
            s = jnp.sum(jnp.where(at != 0, p, 0.0), axis=1, keepdims=True)
            rs = 1.0 / s
            rs = rs * (2.0 - s * rs)
            w = (p * rs) * at
            out = jnp.dot(w, h, preferred_element_type=jnp.float32)
            y = jnp.maximum(out + b2_ref[...], 0.0)
            y_s[pl.ds(g * BATCH + k * N, N), :] = y
            ps = jnp.sum(y, axis=0, keepdims=True)
            qs = jnp.sum(y * y, axis=0, keepdims=True)
            ssum = ps if ssum is None else ssum + ps
            qsum = qs if qsum is None else qsum + qs
        s_s[...] += ssum
        q_s[...] += qsum

    @pl.when(g == T)
    def _gru():
        mean = s_s[...] * (1.0 / ROWS)
        var = q_s[...] * (1.0 / ROWS) - mean * mean
        rstd = lax.rsqrt(var + 1e-5)
        scale = g2_ref[...] * rstd
        shift = be2_ref[...] - mean * scale

        wih1 = wih1_ref[...].T
        whh1 = whh1_ref[...].T
        wih2 = wih2_ref[...].T
        whh2 = whh2_ref[...].T
        bih1 = bih1_ref[...]
        bhh1 = bhh1_ref[...]
        bih2 = bih2_ref[...]
        bhh2 = bhh2_ref[...]

        def gru_step(x, hprev, wih, whh, bi, bh):
            gi = jnp.dot(x, wih, preferred_element_type=jnp.float32) + bi
            gh = jnp.dot(hprev, whh, preferred_element_type=jnp.float32) + bh
            r = jax.nn.sigmoid(gi[:, :H] + gh[:, :H])
            z = jax.nn.sigmoid(gi[:, H:2 * H] + gh[:, H:2 * H])
            n = jnp.tanh(gi[:, 2 * H:] + r * gh[:, 2 * H:])
            return (1.0 - z) * n + z * hprev

        h1 = jnp.zeros((BATCH, H), jnp.float32)
        h2 = jnp.zeros((BATCH, H), jnp.float32)
        for t in range(T):
            x = y_s[t * BATCH:(t + 1) * BATCH, :] * scale + shift
            h1 = gru_step(x, h1, wih1, whh1, bih1, bhh1)
            h2 = gru_step(h1, h2, wih2, whh2, bih2, bhh2)

        z = jnp.concatenate([h1, h2], axis=1)
        u = jnp.dot(z, l1w_ref[...].T, preferred_element_type=jnp.float32)
        u = jnp.maximum(u + l1b_ref[...], 0.0)
        o = jnp.dot(u, l2w_ref[...].T, preferred_element_type=jnp.float32)
        o_ref[...] = jnp.maximum(o + l2b_ref[...], 0.0)


def kernel(X, Ax, W1, as1, ad1, b1, g1, be1, W2, as2, ad2, b2, g2, be2,
           Wih1, Whh1, bih1, bhh1, Wih2, Whh2, bih2, bhh2, L1w, L1b, L2w, L2b):
    f_in = X.shape[-1]
    graph_ix = lambda g: (jnp.minimum(g, T - 1), 0, 0, 0)
    const_ix2 = lambda g: (0, 0)
    out = pl.pallas_call(
        _fused_kernel,
        grid=(T + 1,),
        in_specs=[
            pl.BlockSpec((1, B, N, f_in), graph_ix),
            pl.BlockSpec((1, B, N, N), graph_ix),
            pl.BlockSpec((f_in, H), const_ix2),
            pl.BlockSpec((1, H), const_ix2),
            pl.BlockSpec((H, 1), const_ix2),
            pl.BlockSpec((1, H), const_ix2),
            pl.BlockSpec((1, H), const_ix2),
            pl.BlockSpec((1, H), const_ix2),
            pl.BlockSpec((3 * H, H), const_ix2),
            pl.BlockSpec((3 * H, H), const_ix2),
            pl.BlockSpec((1, 3 * H), const_ix2),
            pl.BlockSpec((1, 3 * H), const_ix2),
            pl.BlockSpec((3 * H, H), const_ix2),
            pl.BlockSpec((3 * H, H), const_ix2),
            pl.BlockSpec((1, 3 * H), const_ix2),
            pl.BlockSpec((1, 3 * H), const_ix2),
            pl.BlockSpec((H, 2 * H), const_ix2),
            pl.BlockSpec((1, H), const_ix2),
            pl.BlockSpec((2, H), const_ix2),
            pl.BlockSpec((1, 2), const_ix2),
        ],
        out_specs=pl.BlockSpec((BATCH, 2), const_ix2),
        out_shape=jax.ShapeDtypeStruct((BATCH, 2), jnp.float32),
        scratch_shapes=[
            pltpu.VMEM((ROWS, H), jnp.float32),
            pltpu.VMEM((1, H), jnp.float32),
            pltpu.VMEM((1, H), jnp.float32),
        ],
        compiler_params=pltpu.CompilerParams(
            vmem_limit_bytes=100 * 1024 * 1024,
        ),
    )(X.reshape(T, B, N, f_in), Ax.reshape(T, B, N, N),
      W2, as2.reshape(1, H), ad2.reshape(H, 1), b2.reshape(1, H),
      g2.reshape(1, H), be2.reshape(1, H),
      Wih1, Whh1, bih1.reshape(1, 3 * H), bhh1.reshape(1, 3 * H),
      Wih2, Whh2, bih2.reshape(1, 3 * H), bhh2.reshape(1, 3 * H),
      L1w, L1b.reshape(1, H), L2w, L2b.reshape(1, 2))

    return out.reshape(B, N, 2)
